# baseline (speedup 1.0000x reference)
import functools

import jax
import jax.numpy as jnp
from jax import lax
from jax.experimental import pallas as pl
from jax.experimental.pallas import tpu as pltpu
from jax.experimental.pallas import tpu_sc as plsc

NUM_NEIGHBORS = 16
DIM = 64
N_PTS = 1024
NA = 4
HID = 64
MULT = 4

_BF = jnp.bfloat16
_F32 = jnp.float32


def _topk_qkv_body(x_ref, x3_ref, f_ref, w_ref, idx_ref, qkv_ref, xg_ref):
    n = N_PTS
    kn = NUM_NEIGHBORS
    b = pl.program_id(0)

    x4 = x_ref[0]
    x3 = x3_ref[0]
    norms_col = jnp.sum(x4 * x4, axis=1, keepdims=True)
    norms_row = jnp.sum(x3 * x3, axis=0, keepdims=True)
    cross = jax.lax.dot_general(
        x4[:, :3].astype(_BF), x3.astype(_BF),
        dimension_numbers=(((1,), (0,)), ((), ())),
        preferred_element_type=_F32)
    dmat = norms_col - 2.0 * cross + norms_row

    iota_row = jax.lax.broadcasted_iota(jnp.int32, (n, n), 1)
    iota_col = jax.lax.broadcasted_iota(jnp.int32, (n, n), 0)
    iota_k = jax.lax.broadcasted_iota(jnp.int32, (n, kn), 1)
    x4t = jnp.concatenate([x3, jnp.zeros((1, n), _F32)], axis=0)
    acc = jnp.zeros((n, kn), dtype=jnp.int32)
    big = jnp.int32(n)
    for t in range(kn):
        m = jnp.min(dmat, axis=1, keepdims=True)
        cand = jnp.where(dmat == m, iota_row, big)
        amin = jnp.min(cand, axis=1)
        acc = jnp.where(iota_k == t, amin[:, None], acc)
        dmat = jnp.where(iota_row == amin[:, None], jnp.float32(jnp.inf), dmat)
        onehot_t = (iota_col == amin[None, :]).astype(_F32)
        xg_ref[0, t] = jnp.dot(x4t, onehot_t,
                               precision=jax.lax.Precision.HIGHEST,
                               preferred_element_type=_F32)
    idx_ref[0] = acc + b * n

    fm = f_ref[0].astype(_BF)
    wm = w_ref[...].astype(_BF)
    qkv_ref[0] = jnp.dot(fm, wm, preferred_element_type=_F32)


def _topk_qkv(xyz_nd4, xyz3, feats_r2, to_qkv_t):
    bsz = xyz_nd4.shape[0]
    n = N_PTS
    return pl.pallas_call(
        _topk_qkv_body,
        grid=(bsz,),
        in_specs=[
            pl.BlockSpec((1, n, 4), lambda b: (b, 0, 0)),
            pl.BlockSpec((1, 3, n), lambda b: (b, 0, 0)),
            pl.BlockSpec((1, n * NA, DIM), lambda b: (b, 0, 0)),
            pl.BlockSpec((DIM, 3 * DIM), lambda b: (0, 0)),
        ],
        out_specs=[
            pl.BlockSpec((1, n, NUM_NEIGHBORS), lambda b: (b, 0, 0)),
            pl.BlockSpec((1, n * NA, 3 * DIM), lambda b: (b, 0, 0)),
            pl.BlockSpec((1, NUM_NEIGHBORS, 4, n), lambda b: (b, 0, 0, 0)),
        ],
        out_shape=[
            jax.ShapeDtypeStruct((bsz, n, NUM_NEIGHBORS), jnp.int32),
            jax.ShapeDtypeStruct((bsz, n * NA, 3 * DIM), _F32),
            jax.ShapeDtypeStruct((bsz, NUM_NEIGHBORS, 4, n), _F32),
        ],
    )(xyz_nd4, xyz3, feats_r2, to_qkv_t)


_SC_CHUNK = 128


def _sc_gather(k_t, v_t, idx_flat):
    r2 = idx_flat.shape[0]
    cdim = k_t.shape[1]
    nworkers = 32
    per_w = r2 // nworkers
    nchunks = per_w // _SC_CHUNK
    mesh = plsc.VectorSubcoreMesh(core_axis_name="c", subcore_axis_name="s")

    @functools.partial(
        pl.kernel,
        mesh=mesh,
        out_type=(
            jax.ShapeDtypeStruct((r2, cdim), _F32),
            jax.ShapeDtypeStruct((r2, cdim), _F32),
        ),
        scratch_types=[
            pltpu.VMEM((_SC_CHUNK,), jnp.int32),
            pltpu.VMEM((_SC_CHUNK, cdim), _F32),
            pltpu.VMEM((_SC_CHUNK, cdim), _F32),
            pltpu.SemaphoreType.DMA,
        ],
    )
    def body(k_hbm, v_hbm, idx_hbm, kg_hbm, vg_hbm, idx_v, kb, vb, sem):
        wid = lax.axis_index("s") * 2 + lax.axis_index("c")
        base = wid * per_w

        def step(c, carry):
            off = pl.multiple_of(base + c * _SC_CHUNK, _SC_CHUNK)
            pltpu.sync_copy(idx_hbm.at[pl.ds(off, _SC_CHUNK)], idx_v)
            pltpu.async_copy(k_hbm.at[idx_v], kb, sem).wait()
            pltpu.sync_copy(kb, kg_hbm.at[pl.ds(off, _SC_CHUNK)])
            pltpu.async_copy(v_hbm.at[idx_v], vb, sem).wait()
            pltpu.sync_copy(vb, vg_hbm.at[pl.ds(off, _SC_CHUNK)])
            return carry

        lax.fori_loop(0, nchunks, step, 0)

    return body(k_t, v_t, idx_flat)


_NB = 128
_N_PIPE = 4


def _dense_body(q_ref, kg_ref, vg_ref, xg_ref, xc_ref,
                wa_ref, wp1_ref, wp2_ref, w1_ref, w2_ref, out_ref):
    nb = _NB
    kn = NUM_NEIGHBORS
    c = NA * DIM

    xc = xc_ref[...]
    xg = xg_ref[...]
    rel = jnp.broadcast_to(xc[:, None, :], (nb, kn, 4)).reshape(nb * kn, 4) - xg

    rot = jnp.dot(rel.astype(_BF), wa_ref[...],
                  preferred_element_type=_F32)
    h = jax.nn.relu(jnp.dot(rot.astype(_BF), wp1_ref[...],
                            preferred_element_type=_F32))
    rpe = jnp.dot(h.astype(_BF), wp2_ref[...],
                  preferred_element_type=_F32)

    qb = q_ref[...]
    q_rep = jnp.broadcast_to(qb[:, None, :], (nb, kn, c)).reshape(nb * kn, c)
    sim0 = q_rep - kg_ref[...] + rpe
    h1 = jax.nn.relu(jnp.dot(sim0.astype(_BF), w1_ref[...],
                             preferred_element_type=_F32))
    sim2 = jnp.dot(h1.astype(_BF), w2_ref[...],
                   preferred_element_type=_F32)

    s3 = sim2.reshape(nb, kn, c)
    m = jnp.max(s3, axis=1, keepdims=True)
    e = jnp.exp(s3 - m)
    den = jnp.sum(e, axis=1, keepdims=True)
    attn = e / den

    vv = (vg_ref[...] + rpe).reshape(nb, kn, c)
    out_ref[...] = jnp.sum(attn * vv, axis=1)


def _dense(q_rows, kg, vg, xg, xc, wa, wp1, wp2, w1, w2):
    r = q_rows.shape[0]
    c = NA * DIM
    kn = NUM_NEIGHBORS
    nblocks = r // _NB
    return pl.pallas_call(
        _dense_body,
        grid=(nblocks,),
        in_specs=[
            pl.BlockSpec((_NB, c), lambda g: (g, 0)),
            pl.BlockSpec((_NB * kn, c), lambda g: (g, 0)),
            pl.BlockSpec((_NB * kn, c), lambda g: (g, 0)),
            pl.BlockSpec((_NB * kn, 4), lambda g: (g, 0)),
            pl.BlockSpec((_NB, 4), lambda g: (g, 0)),
            pl.BlockSpec((4, 16), lambda g: (0, 0)),
            pl.BlockSpec((16, c), lambda g: (0, 0)),
            pl.BlockSpec((c, c), lambda g: (0, 0)),
            pl.BlockSpec((c, MULT * c), lambda g: (0, 0)),
            pl.BlockSpec((MULT * c, c), lambda g: (0, 0)),
        ],
        out_specs=pl.BlockSpec((_NB, c), lambda g: (g, 0)),
        out_shape=jax.ShapeDtypeStruct((r, c), _F32),
    )(q_rows, kg, vg, xg, xc, wa, wp1, wp2, w1, w2)


def _prep_weights(anchors, pos_mlp1, pos_mlp2, attn_mlp1, attn_mlp2):
    dim, hid, c = DIM, HID, NA * DIM
    wa = jnp.pad(anchors.transpose(2, 0, 1),
                 ((0, 1), (0, 0), (0, 1))).reshape(4, NA * 4)
    wp1 = jnp.zeros((NA, 4, NA, hid), _F32)
    wp1 = wp1.at[jnp.arange(NA), :, jnp.arange(NA), :].set(
        jnp.broadcast_to(jnp.pad(pos_mlp1.T, ((0, 1), (0, 0))), (NA, 4, hid)))
    wp1 = wp1.reshape(NA * 4, NA * hid)
    def bdiag(mat):
        p, q = mat.shape
        z = jnp.zeros((NA, p, NA, q), _F32)
        z = z.at[jnp.arange(NA), :, jnp.arange(NA), :].set(
            jnp.broadcast_to(mat, (NA, p, q)))
        return z.reshape(NA * p, NA * q)
    wp2 = bdiag(pos_mlp2.T)
    w1 = bdiag(attn_mlp1.T)
    w2 = bdiag(attn_mlp2.T)
    return (wa.astype(_BF), wp1.astype(_BF), wp2.astype(_BF),
            w1.astype(_BF), w2.astype(_BF))


def kernel(xyz, feats, anchors, to_qkv, pos_mlp1, pos_mlp2, attn_mlp1, attn_mlp2):
    bsz, _, n = xyz.shape
    dim, kn, na = DIM, NUM_NEIGHBORS, NA
    c = na * dim

    xyz_nd4 = jnp.pad(xyz.transpose(0, 2, 1), ((0, 0), (0, 0), (0, 1)))
    feats_r2 = feats.transpose(0, 2, 3, 1).reshape(bsz, n * na, dim)
    to_qkv_t = to_qkv.T

    idx, qkv, xg = _topk_qkv(xyz_nd4, xyz, feats_r2, to_qkv_t)

    qkv_r = qkv.reshape(bsz, n, na, 3 * dim)
    q_rows = qkv_r[..., 0:dim].reshape(bsz * n, c)
    k_t = qkv_r[..., dim:2 * dim].reshape(bsz * n, c)
    v_t = qkv_r[..., 2 * dim:].reshape(bsz * n, c)
    xc_t = xyz_nd4.reshape(bsz * n, 4)
    xg_rows = xg.transpose(0, 3, 1, 2).reshape(bsz * n * kn, 4)

    idx_flat = idx.reshape(bsz * n * kn)
    wa, wp1, wp2, w1, w2 = _prep_weights(anchors, pos_mlp1, pos_mlp2,
                                         attn_mlp1, attn_mlp2)

    nchunk = _N_PIPE
    cs = (bsz * n) // nchunk
    aggs = []
    for ci in range(nchunk):
        sl = slice(ci * cs, (ci + 1) * cs)
        slk = slice(ci * cs * kn, (ci + 1) * cs * kn)
        kg, vg = _sc_gather(k_t, v_t, idx_flat[slk])
        aggs.append(_dense(q_rows[sl], kg, vg, xg_rows[slk], xc_t[sl],
                           wa, wp1, wp2, w1, w2))
    agg = jnp.concatenate(aggs, axis=0)

    return agg.reshape(bsz, n, na, dim).transpose(0, 3, 1, 2)


# speedup vs baseline: 1.0960x; 1.0960x over previous
import functools

import jax
import jax.numpy as jnp
from jax import lax
from jax.experimental import pallas as pl
from jax.experimental.pallas import tpu as pltpu
from jax.experimental.pallas import tpu_sc as plsc

NUM_NEIGHBORS = 16
DIM = 64
N_PTS = 1024
NA = 4
HID = 64
MULT = 4

_BF = jnp.bfloat16
_F32 = jnp.float32


def _topk_qkv_body(x_ref, x3_ref, f_ref, w_ref, idx_ref, qkv_ref, xg_ref):
    n = N_PTS
    kn = NUM_NEIGHBORS
    b = pl.program_id(0)

    x4 = x_ref[0]
    x3 = x3_ref[0]
    norms_col = jnp.sum(x4 * x4, axis=1, keepdims=True)
    norms_row = jnp.sum(x3 * x3, axis=0, keepdims=True)
    cross = jax.lax.dot_general(
        x4[:, :3].astype(_BF), x3.astype(_BF),
        dimension_numbers=(((1,), (0,)), ((), ())),
        preferred_element_type=_F32)
    dmat = norms_col - 2.0 * cross + norms_row

    iota_row = jax.lax.broadcasted_iota(jnp.int32, (n, n), 1)
    iota_col = jax.lax.broadcasted_iota(jnp.int32, (n, n), 0)
    iota_k = jax.lax.broadcasted_iota(jnp.int32, (n, kn), 1)
    x4t = jnp.concatenate([x3, jnp.zeros((1, n), _F32)], axis=0)
    acc = jnp.zeros((n, kn), dtype=jnp.int32)
    big = jnp.int32(n)
    for t in range(kn):
        m = jnp.min(dmat, axis=1, keepdims=True)
        cand = jnp.where(dmat == m, iota_row, big)
        amin = jnp.min(cand, axis=1)
        acc = jnp.where(iota_k == t, amin[:, None], acc)
        dmat = jnp.where(iota_row == amin[:, None], jnp.float32(jnp.inf), dmat)
        onehot_t = (iota_col == amin[None, :]).astype(_F32)
        xg_ref[0, t] = jnp.dot(x4t, onehot_t,
                               precision=jax.lax.Precision.HIGHEST,
                               preferred_element_type=_F32)
    idx_ref[0] = acc + b * n

    fm = f_ref[0].astype(_BF)
    wm = w_ref[...].astype(_BF)
    qkv_ref[0] = jnp.dot(fm, wm, preferred_element_type=_F32)


def _topk_qkv(xyz_nd4, xyz3, feats_r2, to_qkv_t):
    bsz = xyz_nd4.shape[0]
    n = N_PTS
    return pl.pallas_call(
        _topk_qkv_body,
        grid=(bsz,),
        in_specs=[
            pl.BlockSpec((1, n, 4), lambda b: (b, 0, 0)),
            pl.BlockSpec((1, 3, n), lambda b: (b, 0, 0)),
            pl.BlockSpec((1, n * NA, DIM), lambda b: (b, 0, 0)),
            pl.BlockSpec((DIM, 3 * DIM), lambda b: (0, 0)),
        ],
        out_specs=[
            pl.BlockSpec((1, n, NUM_NEIGHBORS), lambda b: (b, 0, 0)),
            pl.BlockSpec((1, n * NA, 3 * DIM), lambda b: (b, 0, 0)),
            pl.BlockSpec((1, NUM_NEIGHBORS, 4, n), lambda b: (b, 0, 0, 0)),
        ],
        out_shape=[
            jax.ShapeDtypeStruct((bsz, n, NUM_NEIGHBORS), jnp.int32),
            jax.ShapeDtypeStruct((bsz, n * NA, 3 * DIM), _F32),
            jax.ShapeDtypeStruct((bsz, NUM_NEIGHBORS, 4, n), _F32),
        ],
    )(xyz_nd4, xyz3, feats_r2, to_qkv_t)


_SC_CHUNK = 128


def _sc_gather(k_t, v_t, idx_flat):
    r2 = idx_flat.shape[0]
    cdim = k_t.shape[1]
    nworkers = 32
    per_w = r2 // nworkers
    nchunks = per_w // _SC_CHUNK
    mesh = plsc.VectorSubcoreMesh(core_axis_name="c", subcore_axis_name="s")

    @functools.partial(
        pl.kernel,
        mesh=mesh,
        out_type=(
            jax.ShapeDtypeStruct((r2, cdim), _F32),
            jax.ShapeDtypeStruct((r2, cdim), _F32),
        ),
        scratch_types=[
            pltpu.VMEM((_SC_CHUNK,), jnp.int32),
            pltpu.VMEM((_SC_CHUNK, cdim), _F32),
            pltpu.VMEM((_SC_CHUNK, cdim), _F32),
            pltpu.SemaphoreType.DMA,
        ],
    )
    def body(k_hbm, v_hbm, idx_hbm, kg_hbm, vg_hbm, idx_v, kb, vb, sem):
        wid = lax.axis_index("s") * 2 + lax.axis_index("c")
        base = wid * per_w

        def step(c, carry):
            off = pl.multiple_of(base + c * _SC_CHUNK, _SC_CHUNK)
            pltpu.sync_copy(idx_hbm.at[pl.ds(off, _SC_CHUNK)], idx_v)
            pltpu.async_copy(k_hbm.at[idx_v], kb, sem).wait()
            pltpu.sync_copy(kb, kg_hbm.at[pl.ds(off, _SC_CHUNK)])
            pltpu.async_copy(v_hbm.at[idx_v], vb, sem).wait()
            pltpu.sync_copy(vb, vg_hbm.at[pl.ds(off, _SC_CHUNK)])
            return carry

        lax.fori_loop(0, nchunks, step, 0)

    return body(k_t, v_t, idx_flat)


_NB = 128
_N_PIPE = 1


def _dense_body(q_ref, kg_ref, vg_ref, xg_ref, xc_ref,
                wa_ref, wp1_ref, wp2_ref, w1_ref, w2_ref, out_ref):
    nb = _NB
    kn = NUM_NEIGHBORS
    c = NA * DIM

    xc = xc_ref[...]
    xg = xg_ref[...]
    rel = jnp.broadcast_to(xc[:, None, :], (nb, kn, 4)).reshape(nb * kn, 4) - xg

    rot = jnp.dot(rel.astype(_BF), wa_ref[...],
                  preferred_element_type=_F32)
    h = jax.nn.relu(jnp.dot(rot.astype(_BF), wp1_ref[...],
                            preferred_element_type=_F32))
    rpe = jnp.dot(h.astype(_BF), wp2_ref[...],
                  preferred_element_type=_F32)

    qb = q_ref[...]
    q_rep = jnp.broadcast_to(qb[:, None, :], (nb, kn, c)).reshape(nb * kn, c)
    sim0 = q_rep - kg_ref[...] + rpe
    h1 = jax.nn.relu(jnp.dot(sim0.astype(_BF), w1_ref[...],
                             preferred_element_type=_F32))
    sim2 = jnp.dot(h1.astype(_BF), w2_ref[...],
                   preferred_element_type=_F32)

    s3 = sim2.reshape(nb, kn, c)
    m = jnp.max(s3, axis=1, keepdims=True)
    e = jnp.exp(s3 - m)
    den = jnp.sum(e, axis=1, keepdims=True)
    attn = e / den

    vv = (vg_ref[...] + rpe).reshape(nb, kn, c)
    out_ref[...] = jnp.sum(attn * vv, axis=1)


def _dense(q_rows, kg, vg, xg, xc, wa, wp1, wp2, w1, w2):
    r = q_rows.shape[0]
    c = NA * DIM
    kn = NUM_NEIGHBORS
    nblocks = r // _NB
    return pl.pallas_call(
        _dense_body,
        grid=(nblocks,),
        in_specs=[
            pl.BlockSpec((_NB, c), lambda g: (g, 0)),
            pl.BlockSpec((_NB * kn, c), lambda g: (g, 0)),
            pl.BlockSpec((_NB * kn, c), lambda g: (g, 0)),
            pl.BlockSpec((_NB * kn, 4), lambda g: (g, 0)),
            pl.BlockSpec((_NB, 4), lambda g: (g, 0)),
            pl.BlockSpec((4, 16), lambda g: (0, 0)),
            pl.BlockSpec((16, c), lambda g: (0, 0)),
            pl.BlockSpec((c, c), lambda g: (0, 0)),
            pl.BlockSpec((c, MULT * c), lambda g: (0, 0)),
            pl.BlockSpec((MULT * c, c), lambda g: (0, 0)),
        ],
        out_specs=pl.BlockSpec((_NB, c), lambda g: (g, 0)),
        out_shape=jax.ShapeDtypeStruct((r, c), _F32),
    )(q_rows, kg, vg, xg, xc, wa, wp1, wp2, w1, w2)


def _prep_weights(anchors, pos_mlp1, pos_mlp2, attn_mlp1, attn_mlp2):
    dim, hid, c = DIM, HID, NA * DIM
    wa = jnp.pad(anchors.transpose(2, 0, 1),
                 ((0, 1), (0, 0), (0, 1))).reshape(4, NA * 4)
    wp1 = jnp.zeros((NA, 4, NA, hid), _F32)
    wp1 = wp1.at[jnp.arange(NA), :, jnp.arange(NA), :].set(
        jnp.broadcast_to(jnp.pad(pos_mlp1.T, ((0, 1), (0, 0))), (NA, 4, hid)))
    wp1 = wp1.reshape(NA * 4, NA * hid)
    def bdiag(mat):
        p, q = mat.shape
        z = jnp.zeros((NA, p, NA, q), _F32)
        z = z.at[jnp.arange(NA), :, jnp.arange(NA), :].set(
            jnp.broadcast_to(mat, (NA, p, q)))
        return z.reshape(NA * p, NA * q)
    wp2 = bdiag(pos_mlp2.T)
    w1 = bdiag(attn_mlp1.T)
    w2 = bdiag(attn_mlp2.T)
    return (wa.astype(_BF), wp1.astype(_BF), wp2.astype(_BF),
            w1.astype(_BF), w2.astype(_BF))


def kernel(xyz, feats, anchors, to_qkv, pos_mlp1, pos_mlp2, attn_mlp1, attn_mlp2):
    bsz, _, n = xyz.shape
    dim, kn, na = DIM, NUM_NEIGHBORS, NA
    c = na * dim

    xyz_nd4 = jnp.pad(xyz.transpose(0, 2, 1), ((0, 0), (0, 0), (0, 1)))
    feats_r2 = feats.transpose(0, 2, 3, 1).reshape(bsz, n * na, dim)
    to_qkv_t = to_qkv.T

    idx, qkv, xg = _topk_qkv(xyz_nd4, xyz, feats_r2, to_qkv_t)

    qkv_r = qkv.reshape(bsz, n, na, 3 * dim)
    q_rows = qkv_r[..., 0:dim].reshape(bsz * n, c)
    k_t = qkv_r[..., dim:2 * dim].reshape(bsz * n, c)
    v_t = qkv_r[..., 2 * dim:].reshape(bsz * n, c)
    xc_t = xyz_nd4.reshape(bsz * n, 4)
    xg_rows = xg.transpose(0, 3, 1, 2).reshape(bsz * n * kn, 4)

    idx_flat = idx.reshape(bsz * n * kn)
    wa, wp1, wp2, w1, w2 = _prep_weights(anchors, pos_mlp1, pos_mlp2,
                                         attn_mlp1, attn_mlp2)

    nchunk = _N_PIPE
    cs = (bsz * n) // nchunk
    aggs = []
    for ci in range(nchunk):
        sl = slice(ci * cs, (ci + 1) * cs)
        slk = slice(ci * cs * kn, (ci + 1) * cs * kn)
        kg, vg = _sc_gather(k_t, v_t, idx_flat[slk])
        aggs.append(_dense(q_rows[sl], kg, vg, xg_rows[slk], xc_t[sl],
                           wa, wp1, wp2, w1, w2))
    agg = jnp.concatenate(aggs, axis=0)

    return agg.reshape(bsz, n, na, dim).transpose(0, 3, 1, 2)


# speedup vs baseline: 1.5268x; 1.3931x over previous
import functools

import jax
import jax.numpy as jnp
from jax import lax
from jax.experimental import pallas as pl
from jax.experimental.pallas import tpu as pltpu
from jax.experimental.pallas import tpu_sc as plsc

NUM_NEIGHBORS = 16
DIM = 64
N_PTS = 1024
NA = 4
HID = 64
MULT = 4

_BF = jnp.bfloat16
_F32 = jnp.float32


def _topk_qkv_body(x_ref, x3_ref, f_ref, w_ref, idx_ref, qkv_ref):
    n = N_PTS
    kn = NUM_NEIGHBORS
    b = pl.program_id(0)

    x4 = x_ref[0]
    x3 = x3_ref[0]
    norms_col = jnp.sum(x4 * x4, axis=1, keepdims=True)
    norms_row = jnp.sum(x3 * x3, axis=0, keepdims=True)
    cross = jax.lax.dot_general(
        x4[:, :3].astype(_BF), x3.astype(_BF),
        dimension_numbers=(((1,), (0,)), ((), ())),
        preferred_element_type=_F32)
    dmat = norms_col - 2.0 * cross + norms_row

    iota_row = jax.lax.broadcasted_iota(jnp.int32, (n, n), 1)
    iota_k = jax.lax.broadcasted_iota(jnp.int32, (n, kn), 1)
    acc = jnp.zeros((n, kn), dtype=jnp.int32)
    big = jnp.int32(n)
    for t in range(kn):
        m = jnp.min(dmat, axis=1, keepdims=True)
        cand = jnp.where(dmat == m, iota_row, big)
        amin = jnp.min(cand, axis=1)
        acc = jnp.where(iota_k == t, amin[:, None], acc)
        dmat = jnp.where(iota_row == amin[:, None], jnp.float32(jnp.inf), dmat)
    idx_ref[0] = acc + b * n

    fm = f_ref[0].astype(_BF)
    wm = w_ref[...].astype(_BF)
    qkv_ref[0] = jnp.dot(fm, wm, preferred_element_type=_F32)


def _topk_qkv(xyz_nd4, xyz3, feats_r2, to_qkv_t):
    bsz = xyz_nd4.shape[0]
    n = N_PTS
    return pl.pallas_call(
        _topk_qkv_body,
        grid=(bsz,),
        in_specs=[
            pl.BlockSpec((1, n, 4), lambda b: (b, 0, 0)),
            pl.BlockSpec((1, 3, n), lambda b: (b, 0, 0)),
            pl.BlockSpec((1, n * NA, DIM), lambda b: (b, 0, 0)),
            pl.BlockSpec((DIM, 3 * DIM), lambda b: (0, 0)),
        ],
        out_specs=[
            pl.BlockSpec((1, n, NUM_NEIGHBORS), lambda b: (b, 0, 0)),
            pl.BlockSpec((1, n * NA, 3 * DIM), lambda b: (b, 0, 0)),
        ],
        out_shape=[
            jax.ShapeDtypeStruct((bsz, n, NUM_NEIGHBORS), jnp.int32),
            jax.ShapeDtypeStruct((bsz, n * NA, 3 * DIM), _F32),
        ],
    )(xyz_nd4, xyz3, feats_r2, to_qkv_t)


_SC_CHUNK = 64


def _sc_gather(kx_t, v_t, idx_flat):
    r2 = idx_flat.shape[0]
    kxdim = kx_t.shape[1]
    cdim = v_t.shape[1]
    nworkers = 32
    per_w = r2 // nworkers
    nchunks = per_w // _SC_CHUNK
    mesh = plsc.VectorSubcoreMesh(core_axis_name="c", subcore_axis_name="s")

    @functools.partial(
        pl.kernel,
        mesh=mesh,
        out_type=(
            jax.ShapeDtypeStruct((r2, kxdim), _F32),
            jax.ShapeDtypeStruct((r2, cdim), _F32),
        ),
        scratch_types=[
            pltpu.VMEM((2, _SC_CHUNK), jnp.int32),
            pltpu.VMEM((2, _SC_CHUNK, kxdim), _F32),
            pltpu.VMEM((2, _SC_CHUNK, cdim), _F32),
            pltpu.SemaphoreType.DMA,
            pltpu.SemaphoreType.DMA,
            pltpu.SemaphoreType.DMA,
            pltpu.SemaphoreType.DMA,
        ],
    )
    def body(kx_hbm, v_hbm, idx_hbm, kxg_hbm, vg_hbm,
             idx_v, kb, vb, semg0, semg1, semw0, semw1):
        wid = lax.axis_index("s") * 2 + lax.axis_index("c")
        base = wid * per_w
        semg = (semg0, semg1)
        semw = (semw0, semw1)

        def off_of(c):
            return pl.multiple_of(base + c * _SC_CHUNK, _SC_CHUNK)

        def issue_gather(c):
            p = c % 2
            pltpu.sync_copy(idx_hbm.at[pl.ds(off_of(c), _SC_CHUNK)],
                            idx_v.at[p])
            hk = pltpu.async_copy(kx_hbm.at[idx_v.at[p]], kb.at[p], semg[p])
            hv = pltpu.async_copy(v_hbm.at[idx_v.at[p]], vb.at[p], semg[p])
            return (hk, hv)

        gh = [None, None]
        wh = [None, None]
        gh[0] = issue_gather(0)
        for c in range(nchunks):
            p = c % 2
            q = (c + 1) % 2
            if c + 1 < nchunks:
                if wh[q] is not None:
                    for h in wh[q]:
                        h.wait()
                    wh[q] = None
                gh[q] = issue_gather(c + 1)
            for h in gh[p]:
                h.wait()
            off = off_of(c)
            w1 = pltpu.async_copy(kb.at[p], kxg_hbm.at[pl.ds(off, _SC_CHUNK)],
                                  semw[p])
            w2 = pltpu.async_copy(vb.at[p], vg_hbm.at[pl.ds(off, _SC_CHUNK)],
                                  semw[p])
            wh[p] = (w1, w2)
        for p in (0, 1):
            if wh[p] is not None:
                for h in wh[p]:
                    h.wait()

    return body(kx_t, v_t, idx_flat)


_NB = 128


def _dense_body(q_ref, kxg_ref, vg_ref, xc_ref,
                wa_ref, wp1_ref, wp2_ref, w1_ref, w2_ref, out_ref):
    nb = _NB
    kn = NUM_NEIGHBORS
    c = NA * DIM

    xc = xc_ref[...]
    xg = kxg_ref[:, c:c + 16]
    rel = jnp.broadcast_to(xc[:, None, :], (nb, kn, 16)).reshape(nb * kn, 16) - xg

    rot = jnp.dot(rel.astype(_BF), wa_ref[...],
                  preferred_element_type=_F32)
    h = jax.nn.relu(jnp.dot(rot.astype(_BF), wp1_ref[...],
                            preferred_element_type=_F32))
    rpe = jnp.dot(h.astype(_BF), wp2_ref[...],
                  preferred_element_type=_F32)

    qb = q_ref[...]
    q_rep = jnp.broadcast_to(qb[:, None, :], (nb, kn, c)).reshape(nb * kn, c)
    sim0 = q_rep - kxg_ref[:, :c] + rpe
    h1 = jax.nn.relu(jnp.dot(sim0.astype(_BF), w1_ref[...],
                             preferred_element_type=_F32))
    sim2 = jnp.dot(h1.astype(_BF), w2_ref[...],
                   preferred_element_type=_F32)

    s3 = sim2.reshape(nb, kn, c)
    m = jnp.max(s3, axis=1, keepdims=True)
    e = jnp.exp(s3 - m)
    den = jnp.sum(e, axis=1, keepdims=True)
    attn = e / den

    vv = (vg_ref[...] + rpe).reshape(nb, kn, c)
    out_ref[...] = jnp.sum(attn * vv, axis=1)


def _dense(q_rows, kxg, vg, xc, wa, wp1, wp2, w1, w2):
    r = q_rows.shape[0]
    c = NA * DIM
    kn = NUM_NEIGHBORS
    nblocks = r // _NB
    return pl.pallas_call(
        _dense_body,
        grid=(nblocks,),
        in_specs=[
            pl.BlockSpec((_NB, c), lambda g: (g, 0)),
            pl.BlockSpec((_NB * kn, c + 128), lambda g: (g, 0)),
            pl.BlockSpec((_NB * kn, c), lambda g: (g, 0)),
            pl.BlockSpec((_NB, 16), lambda g: (g, 0)),
            pl.BlockSpec((16, 16), lambda g: (0, 0)),
            pl.BlockSpec((16, c), lambda g: (0, 0)),
            pl.BlockSpec((c, c), lambda g: (0, 0)),
            pl.BlockSpec((c, MULT * c), lambda g: (0, 0)),
            pl.BlockSpec((MULT * c, c), lambda g: (0, 0)),
        ],
        out_specs=pl.BlockSpec((_NB, c), lambda g: (g, 0)),
        out_shape=jax.ShapeDtypeStruct((r, c), _F32),
    )(q_rows, kxg, vg, xc, wa, wp1, wp2, w1, w2)


def _prep_weights(anchors, pos_mlp1, pos_mlp2, attn_mlp1, attn_mlp2):
    dim, hid, c = DIM, HID, NA * DIM
    wa = jnp.pad(anchors.transpose(2, 0, 1),
                 ((0, 13), (0, 0), (0, 1))).reshape(16, NA * 4)
    wp1 = jnp.zeros((NA, 4, NA, hid), _F32)
    wp1 = wp1.at[jnp.arange(NA), :, jnp.arange(NA), :].set(
        jnp.broadcast_to(jnp.pad(pos_mlp1.T, ((0, 1), (0, 0))), (NA, 4, hid)))
    wp1 = wp1.reshape(NA * 4, NA * hid)
    def bdiag(mat):
        p, q = mat.shape
        z = jnp.zeros((NA, p, NA, q), _F32)
        z = z.at[jnp.arange(NA), :, jnp.arange(NA), :].set(
            jnp.broadcast_to(mat, (NA, p, q)))
        return z.reshape(NA * p, NA * q)
    wp2 = bdiag(pos_mlp2.T)
    w1 = bdiag(attn_mlp1.T)
    w2 = bdiag(attn_mlp2.T)
    return (wa.astype(_BF), wp1.astype(_BF), wp2.astype(_BF),
            w1.astype(_BF), w2.astype(_BF))


def kernel(xyz, feats, anchors, to_qkv, pos_mlp1, pos_mlp2, attn_mlp1, attn_mlp2):
    bsz, _, n = xyz.shape
    dim, kn, na = DIM, NUM_NEIGHBORS, NA
    c = na * dim

    xyz_nd4 = jnp.pad(xyz.transpose(0, 2, 1), ((0, 0), (0, 0), (0, 1)))
    feats_r2 = feats.transpose(0, 2, 3, 1).reshape(bsz, n * na, dim)
    to_qkv_t = to_qkv.T

    idx, qkv = _topk_qkv(xyz_nd4, xyz, feats_r2, to_qkv_t)

    qkv_r = qkv.reshape(bsz, n, na, 3 * dim)
    q_rows = qkv_r[..., 0:dim].reshape(bsz * n, c)
    k_t = qkv_r[..., dim:2 * dim].reshape(bsz * n, c)
    v_t = qkv_r[..., 2 * dim:].reshape(bsz * n, c)
    xyz_flat = xyz_nd4[..., :3].reshape(bsz * n, 3)
    x_t = jnp.pad(xyz_flat, ((0, 0), (0, 13)))
    kx_t = jnp.concatenate([k_t, jnp.pad(xyz_flat, ((0, 0), (0, 125)))], axis=1)

    idx_flat = idx.reshape(bsz * n * kn)
    kxg, vg = _sc_gather(kx_t, v_t, idx_flat)

    wa, wp1, wp2, w1, w2 = _prep_weights(anchors, pos_mlp1, pos_mlp2,
                                         attn_mlp1, attn_mlp2)
    agg = _dense(q_rows, kxg, vg, x_t, wa, wp1, wp2, w1, w2)

    return agg.reshape(bsz, n, na, dim).transpose(0, 3, 1, 2)


# speedup vs baseline: 1.8639x; 1.2208x over previous
import functools

import jax
import jax.numpy as jnp
from jax import lax
from jax.experimental import pallas as pl
from jax.experimental.pallas import tpu as pltpu
from jax.experimental.pallas import tpu_sc as plsc

NUM_NEIGHBORS = 16
DIM = 64
N_PTS = 1024
NA = 4
HID = 64
MULT = 4

_BF = jnp.bfloat16
_F32 = jnp.float32


def _topk_body(x_ref, x3_ref, idx_ref):
    n = N_PTS
    kn = NUM_NEIGHBORS
    b = pl.program_id(0)

    x4 = x_ref[0]
    x3 = x3_ref[0]
    norms_col = jnp.sum(x4 * x4, axis=1, keepdims=True)
    norms_row = jnp.sum(x3 * x3, axis=0, keepdims=True)
    cross = jax.lax.dot_general(
        x4[:, :3].astype(_BF), x3.astype(_BF),
        dimension_numbers=(((1,), (0,)), ((), ())),
        preferred_element_type=_F32)
    dmat = norms_col - 2.0 * cross + norms_row

    iota_row = jax.lax.broadcasted_iota(jnp.int32, (n, n), 1)
    iota_k = jax.lax.broadcasted_iota(jnp.int32, (n, kn), 1)
    acc = jnp.zeros((n, kn), dtype=jnp.int32)
    big = jnp.int32(n)
    for t in range(kn):
        m = jnp.min(dmat, axis=1, keepdims=True)
        cand = jnp.where(dmat == m, iota_row, big)
        amin = jnp.min(cand, axis=1)
        acc = jnp.where(iota_k == t, amin[:, None], acc)
        dmat = jnp.where(iota_row == amin[:, None], jnp.float32(jnp.inf), dmat)
    idx_ref[0] = acc + b * n


def _topk(xyz_nd4, xyz3):
    bsz = xyz_nd4.shape[0]
    n = N_PTS
    return pl.pallas_call(
        _topk_body,
        grid=(bsz,),
        in_specs=[
            pl.BlockSpec((1, n, 4), lambda b: (b, 0, 0)),
            pl.BlockSpec((1, 3, n), lambda b: (b, 0, 0)),
        ],
        out_specs=pl.BlockSpec((1, n, NUM_NEIGHBORS), lambda b: (b, 0, 0)),
        out_shape=jax.ShapeDtypeStruct((bsz, n, NUM_NEIGHBORS), jnp.int32),
    )(xyz_nd4, xyz3)


_SC_CHUNK = 128


def _sc_gather(fx_t, idx_flat):
    r2 = idx_flat.shape[0]
    fdim = fx_t.shape[1]
    nworkers = 32
    per_w = r2 // nworkers
    nchunks = per_w // _SC_CHUNK
    mesh = plsc.VectorSubcoreMesh(core_axis_name="c", subcore_axis_name="s")

    @functools.partial(
        pl.kernel,
        mesh=mesh,
        out_type=jax.ShapeDtypeStruct((r2, fdim), _F32),
        scratch_types=[
            pltpu.VMEM((2, _SC_CHUNK), jnp.int32),
            pltpu.VMEM((2, _SC_CHUNK, fdim), _F32),
            pltpu.SemaphoreType.DMA,
            pltpu.SemaphoreType.DMA,
            pltpu.SemaphoreType.DMA,
            pltpu.SemaphoreType.DMA,
        ],
    )
    def body(fx_hbm, idx_hbm, fxg_hbm, idx_v, fb, semg0, semg1, semw0, semw1):
        wid = lax.axis_index("s") * 2 + lax.axis_index("c")
        base = wid * per_w
        semg = (semg0, semg1)
        semw = (semw0, semw1)

        def off_of(c):
            return pl.multiple_of(base + c * _SC_CHUNK, _SC_CHUNK)

        def issue_gather(c):
            p = c % 2
            pltpu.sync_copy(idx_hbm.at[pl.ds(off_of(c), _SC_CHUNK)],
                            idx_v.at[p])
            return pltpu.async_copy(fx_hbm.at[idx_v.at[p]], fb.at[p], semg[p])

        gh = [None, None]
        wh = [None, None]
        gh[0] = issue_gather(0)
        for c in range(nchunks):
            p = c % 2
            q = (c + 1) % 2
            if c + 1 < nchunks:
                if wh[q] is not None:
                    wh[q].wait()
                    wh[q] = None
                gh[q] = issue_gather(c + 1)
            gh[p].wait()
            wh[p] = pltpu.async_copy(
                fb.at[p], fxg_hbm.at[pl.ds(off_of(c), _SC_CHUNK)], semw[p])
        for p in (0, 1):
            if wh[p] is not None:
                wh[p].wait()

    return body(fx_t, idx_flat)


_NB = 128


def _dense_body(fc_ref, fxg_ref, xc_ref,
                wq_ref, wk_ref, wv_ref,
                wa_ref, wp1_ref, wp2_ref, w1_ref, w2_ref, out_ref):
    nb = _NB
    kn = NUM_NEIGHBORS
    c = NA * DIM

    xc = xc_ref[...]
    xg = fxg_ref[:, c:c + 16]
    rel = jnp.broadcast_to(xc[:, None, :], (nb, kn, 16)).reshape(nb * kn, 16) - xg

    rot = jnp.dot(rel.astype(_BF), wa_ref[...],
                  preferred_element_type=_F32)
    h = jax.nn.relu(jnp.dot(rot.astype(_BF), wp1_ref[...],
                            preferred_element_type=_F32))
    rpe = jnp.dot(h.astype(_BF), wp2_ref[...],
                  preferred_element_type=_F32)

    fg = fxg_ref[:, :c].astype(_BF)
    kg = jnp.dot(fg, wk_ref[...], preferred_element_type=_F32)
    vg = jnp.dot(fg, wv_ref[...], preferred_element_type=_F32)
    qb = jnp.dot(fc_ref[...].astype(_BF), wq_ref[...],
                 preferred_element_type=_F32)

    q_rep = jnp.broadcast_to(qb[:, None, :], (nb, kn, c)).reshape(nb * kn, c)
    sim0 = q_rep - kg + rpe
    h1 = jax.nn.relu(jnp.dot(sim0.astype(_BF), w1_ref[...],
                             preferred_element_type=_F32))
    sim2 = jnp.dot(h1.astype(_BF), w2_ref[...],
                   preferred_element_type=_F32)

    s3 = sim2.reshape(nb, kn, c)
    m = jnp.max(s3, axis=1, keepdims=True)
    e = jnp.exp(s3 - m)
    den = jnp.sum(e, axis=1, keepdims=True)
    attn = e / den

    vv = (vg + rpe).reshape(nb, kn, c)
    out_ref[...] = jnp.sum(attn * vv, axis=1)


def _dense(fc, fxg, xc, wq, wk, wv, wa, wp1, wp2, w1, w2):
    r = fc.shape[0]
    c = NA * DIM
    kn = NUM_NEIGHBORS
    nblocks = r // _NB
    return pl.pallas_call(
        _dense_body,
        grid=(nblocks,),
        in_specs=[
            pl.BlockSpec((_NB, c), lambda g: (g, 0)),
            pl.BlockSpec((_NB * kn, c + 128), lambda g: (g, 0)),
            pl.BlockSpec((_NB, 16), lambda g: (g, 0)),
            pl.BlockSpec((c, c), lambda g: (0, 0)),
            pl.BlockSpec((c, c), lambda g: (0, 0)),
            pl.BlockSpec((c, c), lambda g: (0, 0)),
            pl.BlockSpec((16, 16), lambda g: (0, 0)),
            pl.BlockSpec((16, c), lambda g: (0, 0)),
            pl.BlockSpec((c, c), lambda g: (0, 0)),
            pl.BlockSpec((c, MULT * c), lambda g: (0, 0)),
            pl.BlockSpec((MULT * c, c), lambda g: (0, 0)),
        ],
        out_specs=pl.BlockSpec((_NB, c), lambda g: (g, 0)),
        out_shape=jax.ShapeDtypeStruct((r, c), _F32),
    )(fc, fxg, xc, wq, wk, wv, wa, wp1, wp2, w1, w2)


def _prep_weights(anchors, to_qkv, pos_mlp1, pos_mlp2, attn_mlp1, attn_mlp2):
    dim, hid, c = DIM, HID, NA * DIM
    wa = jnp.pad(anchors.transpose(2, 0, 1),
                 ((0, 13), (0, 0), (0, 1))).reshape(16, NA * 4)
    wp1 = jnp.zeros((NA, 4, NA, hid), _F32)
    wp1 = wp1.at[jnp.arange(NA), :, jnp.arange(NA), :].set(
        jnp.broadcast_to(jnp.pad(pos_mlp1.T, ((0, 1), (0, 0))), (NA, 4, hid)))
    wp1 = wp1.reshape(NA * 4, NA * hid)
    def bdiag(mat):
        p, q = mat.shape
        z = jnp.zeros((NA, p, NA, q), _F32)
        z = z.at[jnp.arange(NA), :, jnp.arange(NA), :].set(
            jnp.broadcast_to(mat, (NA, p, q)))
        return z.reshape(NA * p, NA * q)
    wq = bdiag(to_qkv[0:dim].T)
    wk = bdiag(to_qkv[dim:2 * dim].T)
    wv = bdiag(to_qkv[2 * dim:].T)
    wp2 = bdiag(pos_mlp2.T)
    w1 = bdiag(attn_mlp1.T)
    w2 = bdiag(attn_mlp2.T)
    return (wq.astype(_BF), wk.astype(_BF), wv.astype(_BF),
            wa.astype(_BF), wp1.astype(_BF), wp2.astype(_BF),
            w1.astype(_BF), w2.astype(_BF))


def kernel(xyz, feats, anchors, to_qkv, pos_mlp1, pos_mlp2, attn_mlp1, attn_mlp2):
    bsz, _, n = xyz.shape
    dim, kn, na = DIM, NUM_NEIGHBORS, NA
    c = na * dim

    xyz_nd4 = jnp.pad(xyz.transpose(0, 2, 1), ((0, 0), (0, 0), (0, 1)))
    feats_rows = feats.transpose(0, 2, 3, 1).reshape(bsz * n, c)

    idx = _topk(xyz_nd4, xyz)

    xyz_flat = xyz_nd4[..., :3].reshape(bsz * n, 3)
    x_t = jnp.pad(xyz_flat, ((0, 0), (0, 13)))
    fx_t = jnp.concatenate(
        [feats_rows, jnp.pad(xyz_flat, ((0, 0), (0, 125)))], axis=1)

    idx_flat = idx.reshape(bsz * n * kn)
    fxg = _sc_gather(fx_t, idx_flat)

    wq, wk, wv, wa, wp1, wp2, w1, w2 = _prep_weights(
        anchors, to_qkv, pos_mlp1, pos_mlp2, attn_mlp1, attn_mlp2)
    agg = _dense(feats_rows, fxg, x_t, wq, wk, wv, wa, wp1, wp2, w1, w2)

    return agg.reshape(bsz, n, na, dim).transpose(0, 3, 1, 2)


# speedup vs baseline: 1.8650x; 1.0006x over previous
import functools

import jax
import jax.numpy as jnp
from jax import lax
from jax.experimental import pallas as pl
from jax.experimental.pallas import tpu as pltpu
from jax.experimental.pallas import tpu_sc as plsc

NUM_NEIGHBORS = 16
DIM = 64
N_PTS = 1024
NA = 4
HID = 64
MULT = 4

_BF = jnp.bfloat16
_F32 = jnp.float32


def _topk_body(x_ref, x3_ref, idx_ref):
    n = N_PTS
    kn = NUM_NEIGHBORS
    b = pl.program_id(0)

    x4 = x_ref[0]
    x3 = x3_ref[0]
    norms_col = jnp.sum(x4 * x4, axis=1, keepdims=True)
    norms_row = jnp.sum(x3 * x3, axis=0, keepdims=True)
    cross = jax.lax.dot_general(
        x4[:, :3].astype(_BF), x3.astype(_BF),
        dimension_numbers=(((1,), (0,)), ((), ())),
        preferred_element_type=_F32)
    dmat = norms_col - 2.0 * cross + norms_row

    iota_row = jax.lax.broadcasted_iota(jnp.int32, (n, n), 1)
    iota_k = jax.lax.broadcasted_iota(jnp.int32, (n, kn), 1)
    acc = jnp.zeros((n, kn), dtype=jnp.int32)
    big = jnp.int32(n)
    for t in range(kn):
        m = jnp.min(dmat, axis=1, keepdims=True)
        cand = jnp.where(dmat == m, iota_row, big)
        amin = jnp.min(cand, axis=1)
        acc = jnp.where(iota_k == t, amin[:, None], acc)
        dmat = jnp.where(iota_row == amin[:, None], jnp.float32(jnp.inf), dmat)
    idx_ref[0] = acc + b * n


def _topk(xyz_nd4, xyz3):
    bsz = xyz_nd4.shape[0]
    n = N_PTS
    return pl.pallas_call(
        _topk_body,
        grid=(bsz,),
        in_specs=[
            pl.BlockSpec((1, n, 4), lambda b: (b, 0, 0)),
            pl.BlockSpec((1, 3, n), lambda b: (b, 0, 0)),
        ],
        out_specs=pl.BlockSpec((1, n, NUM_NEIGHBORS), lambda b: (b, 0, 0)),
        out_shape=jax.ShapeDtypeStruct((bsz, n, NUM_NEIGHBORS), jnp.int32),
    )(xyz_nd4, xyz3)


_SC_CHUNK = 128


def _sc_gather(fx_t, idx_flat):
    r2 = idx_flat.shape[0]
    fdim = fx_t.shape[1]
    nworkers = 32
    per_w = r2 // nworkers
    nchunks = per_w // _SC_CHUNK
    mesh = plsc.VectorSubcoreMesh(core_axis_name="c", subcore_axis_name="s")

    @functools.partial(
        pl.kernel,
        mesh=mesh,
        out_type=jax.ShapeDtypeStruct((r2, fdim), _F32),
        scratch_types=[
            pltpu.VMEM((2, _SC_CHUNK), jnp.int32),
            pltpu.VMEM((2, _SC_CHUNK, fdim), _F32),
            pltpu.SemaphoreType.DMA,
            pltpu.SemaphoreType.DMA,
            pltpu.SemaphoreType.DMA,
            pltpu.SemaphoreType.DMA,
        ],
    )
    def body(fx_hbm, idx_hbm, fxg_hbm, idx_v, fb, semg0, semg1, semw0, semw1):
        wid = lax.axis_index("s") * 2 + lax.axis_index("c")
        base = wid * per_w
        semg = (semg0, semg1)
        semw = (semw0, semw1)

        def off_of(c):
            return pl.multiple_of(base + c * _SC_CHUNK, _SC_CHUNK)

        def issue_gather(c):
            p = c % 2
            pltpu.sync_copy(idx_hbm.at[pl.ds(off_of(c), _SC_CHUNK)],
                            idx_v.at[p])
            return pltpu.async_copy(fx_hbm.at[idx_v.at[p]], fb.at[p], semg[p])

        gh = [None, None]
        wh = [None, None]
        gh[0] = issue_gather(0)
        for c in range(nchunks):
            p = c % 2
            q = (c + 1) % 2
            if c + 1 < nchunks:
                if wh[q] is not None:
                    wh[q].wait()
                    wh[q] = None
                gh[q] = issue_gather(c + 1)
            gh[p].wait()
            wh[p] = pltpu.async_copy(
                fb.at[p], fxg_hbm.at[pl.ds(off_of(c), _SC_CHUNK)], semw[p])
        for p in (0, 1):
            if wh[p] is not None:
                wh[p].wait()

    return body(fx_t, idx_flat)


_NB = 256


def _dense_body(fc_ref, fxg_ref, xc_ref,
                wq_ref, wk_ref, wv_ref,
                wa_ref, wp1_ref, wp2_ref, w1_ref, w2_ref, out_ref):
    nb = _NB
    kn = NUM_NEIGHBORS
    c = NA * DIM

    xc = xc_ref[...]
    xg = fxg_ref[:, c:c + 16]
    rel = jnp.broadcast_to(xc[:, None, :], (nb, kn, 16)).reshape(nb * kn, 16) - xg

    rot = jnp.dot(rel.astype(_BF), wa_ref[...],
                  preferred_element_type=_F32)
    h = jax.nn.relu(jnp.dot(rot.astype(_BF), wp1_ref[...],
                            preferred_element_type=_F32))
    rpe = jnp.dot(h.astype(_BF), wp2_ref[...],
                  preferred_element_type=_F32)

    fg = fxg_ref[:, :c].astype(_BF)
    kg = jnp.dot(fg, wk_ref[...], preferred_element_type=_F32)
    vg = jnp.dot(fg, wv_ref[...], preferred_element_type=_F32)
    qb = jnp.dot(fc_ref[...].astype(_BF), wq_ref[...],
                 preferred_element_type=_F32)

    q_rep = jnp.broadcast_to(qb[:, None, :], (nb, kn, c)).reshape(nb * kn, c)
    sim0 = q_rep - kg + rpe
    h1 = jax.nn.relu(jnp.dot(sim0.astype(_BF), w1_ref[...],
                             preferred_element_type=_F32))
    sim2 = jnp.dot(h1.astype(_BF), w2_ref[...],
                   preferred_element_type=_F32)

    s3 = sim2.reshape(nb, kn, c)
    m = jnp.max(s3, axis=1, keepdims=True)
    e = jnp.exp(s3 - m)
    den = jnp.sum(e, axis=1, keepdims=True)
    attn = e / den

    vv = (vg + rpe).reshape(nb, kn, c)
    out_ref[...] = jnp.sum(attn * vv, axis=1)


def _dense(fc, fxg, xc, wq, wk, wv, wa, wp1, wp2, w1, w2):
    r = fc.shape[0]
    c = NA * DIM
    kn = NUM_NEIGHBORS
    nblocks = r // _NB
    return pl.pallas_call(
        _dense_body,
        grid=(nblocks,),
        in_specs=[
            pl.BlockSpec((_NB, c), lambda g: (g, 0)),
            pl.BlockSpec((_NB * kn, c + 128), lambda g: (g, 0)),
            pl.BlockSpec((_NB, 16), lambda g: (g, 0)),
            pl.BlockSpec((c, c), lambda g: (0, 0)),
            pl.BlockSpec((c, c), lambda g: (0, 0)),
            pl.BlockSpec((c, c), lambda g: (0, 0)),
            pl.BlockSpec((16, 16), lambda g: (0, 0)),
            pl.BlockSpec((16, c), lambda g: (0, 0)),
            pl.BlockSpec((c, c), lambda g: (0, 0)),
            pl.BlockSpec((c, MULT * c), lambda g: (0, 0)),
            pl.BlockSpec((MULT * c, c), lambda g: (0, 0)),
        ],
        out_specs=pl.BlockSpec((_NB, c), lambda g: (g, 0)),
        out_shape=jax.ShapeDtypeStruct((r, c), _F32),
    )(fc, fxg, xc, wq, wk, wv, wa, wp1, wp2, w1, w2)


def _prep_weights(anchors, to_qkv, pos_mlp1, pos_mlp2, attn_mlp1, attn_mlp2):
    dim, hid, c = DIM, HID, NA * DIM
    wa = jnp.pad(anchors.transpose(2, 0, 1),
                 ((0, 13), (0, 0), (0, 1))).reshape(16, NA * 4)
    wp1 = jnp.zeros((NA, 4, NA, hid), _F32)
    wp1 = wp1.at[jnp.arange(NA), :, jnp.arange(NA), :].set(
        jnp.broadcast_to(jnp.pad(pos_mlp1.T, ((0, 1), (0, 0))), (NA, 4, hid)))
    wp1 = wp1.reshape(NA * 4, NA * hid)
    def bdiag(mat):
        p, q = mat.shape
        z = jnp.zeros((NA, p, NA, q), _F32)
        z = z.at[jnp.arange(NA), :, jnp.arange(NA), :].set(
            jnp.broadcast_to(mat, (NA, p, q)))
        return z.reshape(NA * p, NA * q)
    wq = bdiag(to_qkv[0:dim].T)
    wk = bdiag(to_qkv[dim:2 * dim].T)
    wv = bdiag(to_qkv[2 * dim:].T)
    wp2 = bdiag(pos_mlp2.T)
    w1 = bdiag(attn_mlp1.T)
    w2 = bdiag(attn_mlp2.T)
    return (wq.astype(_BF), wk.astype(_BF), wv.astype(_BF),
            wa.astype(_BF), wp1.astype(_BF), wp2.astype(_BF),
            w1.astype(_BF), w2.astype(_BF))


def kernel(xyz, feats, anchors, to_qkv, pos_mlp1, pos_mlp2, attn_mlp1, attn_mlp2):
    bsz, _, n = xyz.shape
    dim, kn, na = DIM, NUM_NEIGHBORS, NA
    c = na * dim

    xyz_nd4 = jnp.pad(xyz.transpose(0, 2, 1), ((0, 0), (0, 0), (0, 1)))
    feats_rows = feats.transpose(0, 2, 3, 1).reshape(bsz * n, c)

    idx = _topk(xyz_nd4, xyz)

    xyz_flat = xyz_nd4[..., :3].reshape(bsz * n, 3)
    x_t = jnp.pad(xyz_flat, ((0, 0), (0, 13)))
    fx_t = jnp.concatenate(
        [feats_rows, jnp.pad(xyz_flat, ((0, 0), (0, 125)))], axis=1)

    idx_flat = idx.reshape(bsz * n * kn)
    fxg = _sc_gather(fx_t, idx_flat)

    wq, wk, wv, wa, wp1, wp2, w1, w2 = _prep_weights(
        anchors, to_qkv, pos_mlp1, pos_mlp2, attn_mlp1, attn_mlp2)
    agg = _dense(feats_rows, fxg, x_t, wq, wk, wv, wa, wp1, wp2, w1, w2)

    return agg.reshape(bsz, n, na, dim).transpose(0, 3, 1, 2)
